# Initial kernel scaffold; baseline (speedup 1.0000x reference)
import jax
import jax.numpy as jnp
from jax.experimental import pallas as pl


def kernel(logits, uniform):
    raise NotImplementedError("write your pallas kernel here")


# speedup vs baseline: 3.3161x; 3.3161x over previous
import functools

import jax
import jax.numpy as jnp
from jax.experimental import pallas as pl
from jax.experimental.pallas import tpu as pltpu

TAU = 0.5
K_SEL = 10
B = 128
D = 32768
G = 256
L = 128
NEG_INF = float("-inf")


def _row_body(logits_ref, unif_ref, samples_ref, disc_ref):
    l = logits_ref[0]
    lmax = jnp.max(l)
    e_logit = jnp.exp((l - lmax) * 2.0)

    acc = jnp.zeros_like(l)
    for k in range(K_SEL):
        w = jnp.log(unif_ref[0, k])
        e = e_logit / (w * w)
        s = jnp.sum(e)
        acc = jnp.maximum(acc, e * (1.0 / s))
    samples_ref[0] = acc

    x = l
    remaining = jnp.int32(K_SEL)
    thr = jnp.float32(NEG_INF)
    for _ in range(K_SEL):
        m = jnp.max(x)
        thr = jnp.where(remaining > 0, m, thr)
        c = jnp.sum(jnp.where(x == m, 1, 0).astype(jnp.int32))
        remaining = jnp.where(remaining > 0, remaining - c, remaining)
        x = jnp.where(x == m, NEG_INF, x)
    disc_ref[0] = (l >= thr).astype(jnp.float32)


@jax.jit
def kernel(logits, uniform):
    logits_r = logits.reshape(B, G, L)
    uniform_r = uniform.reshape(B, K_SEL, G, L)
    samples, disc = pl.pallas_call(
        _row_body,
        grid=(B,),
        in_specs=[
            pl.BlockSpec((1, G, L), lambda b: (b, 0, 0)),
            pl.BlockSpec((1, K_SEL, G, L), lambda b: (b, 0, 0, 0)),
        ],
        out_specs=[
            pl.BlockSpec((1, G, L), lambda b: (b, 0, 0)),
            pl.BlockSpec((1, G, L), lambda b: (b, 0, 0)),
        ],
        out_shape=[
            jax.ShapeDtypeStruct((B, G, L), jnp.float32),
            jax.ShapeDtypeStruct((B, G, L), jnp.float32),
        ],
    )(logits_r, uniform_r)
    return samples.reshape(B, D), disc.reshape(B, D)
